# baseline (speedup 1.0000x reference)
import functools

import jax
import jax.numpy as jnp
from jax import lax
from jax.experimental import pallas as pl
from jax.experimental.pallas import tpu as pltpu
from jax.experimental.pallas import tpu_sc as plsc

_NC = 2
_NS = 16
_NW = _NC * _NS
_SUB = 128


@functools.cache
def _build(S, T, D):
    B = S * T
    b_per_w = B // _NW
    s_per_w = S // _NW
    n_chunks = s_per_w // 2
    n_pairs = n_chunks // 2
    subs = []
    off = 0
    while off < T:
        ln = min(_SUB, T - off)
        subs.append((off, ln))
        off += ln
    mesh = plsc.VectorSubcoreMesh(core_axis_name="c", subcore_axis_name="s")

    @functools.partial(
        pl.kernel,
        out_type=jax.ShapeDtypeStruct((S, T, D), jnp.float32),
        mesh=mesh,
        scratch_types=[
            pltpu.VMEM((b_per_w,), jnp.int32),
            pltpu.VMEM((2, T, D), jnp.float32),
            pltpu.VMEM((2, T, D), jnp.float32),
            pltpu.SemaphoreType.DMA,
            pltpu.SemaphoreType.DMA,
            pltpu.SemaphoreType.DMA,
            pltpu.SemaphoreType.DMA,
        ],
        compiler_params=pltpu.CompilerParams(use_tc_tiling_on_sc=False),
    )
    def gather_kernel(table_hbm, flat_ev_hbm, out_hbm,
                      idx_v, buf0, buf1, gsem0, gsem1, wsem0, wsem1):
        wid = lax.axis_index("s") * _NC + lax.axis_index("c")
        base = pl.multiple_of(wid * b_per_w, 8)
        srow = pl.multiple_of(wid * s_per_w, 2)
        pltpu.sync_copy(flat_ev_hbm.at[pl.ds(base, b_per_w)], idx_v)

        def fire_gather(c, buf, sem):
            for k in range(2):
                for (o, ln) in subs:
                    ioff = pl.multiple_of(c * 2 * T + k * T + o, 8)
                    pltpu.async_copy(
                        table_hbm.at[idx_v.at[pl.ds(ioff, ln)]],
                        buf.at[k, pl.ds(o, ln), :],
                        sem,
                    )

        def wait_gather(buf, sem):
            pltpu.make_async_copy(out_hbm.at[pl.ds(0, 2), :, :], buf, sem).wait()

        def fire_write(buf, c, sem):
            soff = pl.multiple_of(srow + c * 2, 2)
            pltpu.async_copy(buf, out_hbm.at[pl.ds(soff, 2), :, :], sem)

        def wait_write(buf, sem):
            pltpu.make_async_copy(buf, out_hbm.at[pl.ds(0, 2), :, :], sem).wait()

        fire_gather(0, buf0, gsem0)
        fire_gather(1, buf1, gsem1)

        def body(i, carry):
            wait_gather(buf0, gsem0)
            fire_write(buf0, 2 * i, wsem0)
            wait_gather(buf1, gsem1)
            fire_write(buf1, 2 * i + 1, wsem1)

            @pl.when(i < n_pairs - 1)
            def _refill():
                wait_write(buf0, wsem0)
                fire_gather(2 * i + 2, buf0, gsem0)
                wait_write(buf1, wsem1)
                fire_gather(2 * i + 3, buf1, gsem1)

            return carry

        lax.fori_loop(0, n_pairs, body, 0)
        wait_write(buf0, wsem0)
        wait_write(buf1, wsem1)

    return gather_kernel


def kernel(event, table):
    S, T = event.shape
    D = table.shape[1]
    flat = event.reshape(S * T)
    return _build(S, T, D)(table, flat)
